# baseline (speedup 1.0000x reference)
import jax
import jax.numpy as jnp
from jax import lax
from jax.experimental import pallas as pl

N_SAMPLES = 4
F = 64
S = 2048
I = F * S
BW = 16384
NB = I // BW


def _row_kernel(x_ref, o_ref):
    j = pl.program_id(0)
    cb = pl.program_id(1)
    row = x_ref[0, 0, :]
    o_ref[0] = jnp.broadcast_to(row[None, :], (N_SAMPLES, BW))

    @pl.when(cb == (j * S) // BW)
    def _():
        off = (j * S) % BW
        chunk2 = x_ref[0, :, pl.ds(off, S)]
        m = jnp.max(chunk2)
        idx = lax.broadcasted_iota(jnp.int32, (1, S), 1)
        a = jnp.min(jnp.where(chunk2 == m, idx, S))
        cols = lax.broadcasted_iota(jnp.int32, (N_SAMPLES, S), 1)
        prow = lax.broadcasted_iota(jnp.int32, (N_SAMPLES, S), 0)
        t = a + prow
        t = jnp.where(t >= S, t - S, t)
        o_ref[0, :, pl.ds(off, S)] = jnp.where(cols == t, m, jnp.float32(0.0))


def kernel(x):
    f, i = x.shape
    x3 = x.reshape(f, 1, i)
    return pl.pallas_call(
        _row_kernel,
        grid=(f, NB),
        in_specs=[pl.BlockSpec((1, 1, BW), lambda j, c: (j, 0, c))],
        out_specs=pl.BlockSpec((1, N_SAMPLES, BW), lambda j, c: (j, 0, c)),
        out_shape=jax.ShapeDtypeStruct((f, N_SAMPLES, i), x.dtype),
    )(x3)


# speedup vs baseline: 4.0005x; 4.0005x over previous
import functools

import jax
import jax.numpy as jnp
from jax import lax
from jax.experimental import pallas as pl
from jax.experimental.pallas import tpu as pltpu
from jax.experimental.pallas import tpu_sc as plsc

F = 64
S = 2048
N_SAMPLES = 4
I = F * S
L = 16
ROWS_PER_W = 2
PIECE = 16384
NPIECE = I // PIECE
NBUF = 6

_mesh = plsc.VectorSubcoreMesh(core_axis_name="c", subcore_axis_name="s")


@functools.partial(
    pl.kernel,
    out_type=jax.ShapeDtypeStruct((F, N_SAMPLES, I), jnp.float32),
    mesh=_mesh,
    scratch_types=[
        [pltpu.VMEM((PIECE,), jnp.float32)] * NBUF,
        pltpu.VMEM((S,), jnp.float32),
        pltpu.VMEM((ROWS_PER_W, N_SAMPLES, S), jnp.float32),
        pltpu.SemaphoreType.DMA,
        [pltpu.SemaphoreType.DMA] * NBUF,
        pltpu.SemaphoreType.DMA,
    ],
)
def _sc_kernel(x_hbm, out_hbm, ring_v, chunk_v, buf_v, sem_r, sem_ws, sem_d):
    cid = lax.axis_index("c")
    sid = lax.axis_index("s")
    wid = sid * 2 + cid
    lane = lax.iota(jnp.int32, L)

    rows = [wid * ROWS_PER_W + r for r in range(ROWS_PER_W)]

    def zfill(k, _):
        buf_v[0, 0, pl.ds(k * L, L)] = jnp.zeros((L,), jnp.float32)
        buf_v[0, 1, pl.ds(k * L, L)] = jnp.zeros((L,), jnp.float32)
        buf_v[0, 2, pl.ds(k * L, L)] = jnp.zeros((L,), jnp.float32)
        buf_v[0, 3, pl.ds(k * L, L)] = jnp.zeros((L,), jnp.float32)
        buf_v[1, 0, pl.ds(k * L, L)] = jnp.zeros((L,), jnp.float32)
        buf_v[1, 1, pl.ds(k * L, L)] = jnp.zeros((L,), jnp.float32)
        buf_v[1, 2, pl.ds(k * L, L)] = jnp.zeros((L,), jnp.float32)
        buf_v[1, 3, pl.ds(k * L, L)] = jnp.zeros((L,), jnp.float32)
        return 0

    lax.fori_loop(0, S // L, zfill, 0)

    for r, j in enumerate(rows):
        pltpu.sync_copy(x_hbm.at[j, pl.ds(j * S, S)], chunk_v)

        def body(i, carry):
            best, bidx = carry
            v = chunk_v[pl.ds(i * L, L)]
            upd = v > best
            best = jnp.where(upd, v, best)
            bidx = jnp.where(upd, i * L + lane, bidx)
            return best, bidx

        best, bidx = lax.fori_loop(
            0, S // L, body,
            (jnp.full((L,), -jnp.inf, jnp.float32), jnp.zeros((L,), jnp.int32)),
        )
        m = jnp.float32(-jnp.inf)
        a = jnp.int32(S)
        for l in range(L):
            vl = best[l]
            il = bidx[l]
            take = (vl > m) | ((vl == m) & (il < a))
            m = jnp.where(take, vl, m)
            a = jnp.where(take, il, a)

        for p in range(N_SAMPLES):
            q = a + p
            q = jnp.where(q >= S, q - S, q)
            blk = (q // L) * L
            buf_v[r, p, pl.ds(blk, L)] = jnp.where(
                lane == q - blk, m, jnp.float32(0.0))

    pieces = [(j, pi) for j in rows for pi in range(NPIECE)]
    reads = {}
    writes = {}

    def issue_read(i):
        j, pi = pieces[i]
        b = i % NBUF
        reads[i] = pltpu.async_copy(
            x_hbm.at[j, pl.ds(pi * PIECE, PIECE)], ring_v[b], sem_r)

    def issue_writes(i):
        j, pi = pieces[i]
        b = i % NBUF
        writes[i] = [
            pltpu.async_copy(
                ring_v[b], out_hbm.at[j, p, pl.ds(pi * PIECE, PIECE)],
                sem_ws[b])
            for p in range(N_SAMPLES)
        ]

    for i in range(min(NBUF, len(pieces))):
        issue_read(i)
    for i in range(len(pieces)):
        reads[i].wait()
        issue_writes(i)
        if i + NBUF < len(pieces):
            for c in writes[i]:
                c.wait()
            issue_read(i + NBUF)
    for i in range(len(pieces) - NBUF, len(pieces)):
        for c in writes[i]:
            c.wait()

    diag = []
    for r, j in enumerate(rows):
        for p in range(N_SAMPLES):
            diag.append(pltpu.async_copy(
                buf_v.at[r, p], out_hbm.at[j, p, pl.ds(j * S, S)], sem_d))
    for c in diag:
        c.wait()


def kernel(x):
    return _sc_kernel(x)


# speedup vs baseline: 4.2873x; 1.0717x over previous
import functools

import jax
import jax.numpy as jnp
from jax import lax
from jax.experimental import pallas as pl
from jax.experimental.pallas import tpu as pltpu
from jax.experimental.pallas import tpu_sc as plsc

F = 64
S = 2048
N_SAMPLES = 4
I = F * S
L = 16
ROWS_PER_W = 2
PIECE = 32768
NPIECE = I // PIECE
NBUF = 3

_mesh = plsc.VectorSubcoreMesh(core_axis_name="c", subcore_axis_name="s")


@functools.partial(
    pl.kernel,
    out_type=jax.ShapeDtypeStruct((F, N_SAMPLES, I), jnp.float32),
    mesh=_mesh,
    scratch_types=[
        [pltpu.VMEM((PIECE,), jnp.float32)] * NBUF,
        pltpu.VMEM((S,), jnp.float32),
        pltpu.VMEM((ROWS_PER_W, N_SAMPLES, S), jnp.float32),
        pltpu.SemaphoreType.DMA,
        [pltpu.SemaphoreType.DMA] * NBUF,
        pltpu.SemaphoreType.DMA,
    ],
)
def _sc_kernel(x_hbm, out_hbm, ring_v, chunk_v, buf_v, sem_r, sem_ws, sem_d):
    cid = lax.axis_index("c")
    sid = lax.axis_index("s")
    wid = sid * 2 + cid
    lane = lax.iota(jnp.int32, L)

    rows = [wid * ROWS_PER_W + r for r in range(ROWS_PER_W)]

    pieces = [(j, pi) for j in rows for pi in range(NPIECE)]
    reads = {}
    writes = {}

    def issue_read(i):
        j, pi = pieces[i]
        b = i % NBUF
        reads[i] = pltpu.async_copy(
            x_hbm.at[j, pl.ds(pi * PIECE, PIECE)], ring_v[b], sem_r)

    def issue_writes(i):
        j, pi = pieces[i]
        b = i % NBUF
        writes[i] = [
            pltpu.async_copy(
                ring_v[b], out_hbm.at[j, p, pl.ds(pi * PIECE, PIECE)],
                sem_ws[b])
            for p in range(N_SAMPLES)
        ]

    for i in range(min(NBUF, len(pieces))):
        issue_read(i)
    reads[0].wait()
    issue_writes(0)

    def zfill(k, _):
        buf_v[0, 0, pl.ds(k * L, L)] = jnp.zeros((L,), jnp.float32)
        buf_v[0, 1, pl.ds(k * L, L)] = jnp.zeros((L,), jnp.float32)
        buf_v[0, 2, pl.ds(k * L, L)] = jnp.zeros((L,), jnp.float32)
        buf_v[0, 3, pl.ds(k * L, L)] = jnp.zeros((L,), jnp.float32)
        buf_v[1, 0, pl.ds(k * L, L)] = jnp.zeros((L,), jnp.float32)
        buf_v[1, 1, pl.ds(k * L, L)] = jnp.zeros((L,), jnp.float32)
        buf_v[1, 2, pl.ds(k * L, L)] = jnp.zeros((L,), jnp.float32)
        buf_v[1, 3, pl.ds(k * L, L)] = jnp.zeros((L,), jnp.float32)
        return 0

    lax.fori_loop(0, S // L, zfill, 0)

    for r, j in enumerate(rows):
        pltpu.sync_copy(x_hbm.at[j, pl.ds(j * S, S)], chunk_v)

        def body(i, carry):
            best, bidx = carry
            v = chunk_v[pl.ds(i * L, L)]
            upd = v > best
            best = jnp.where(upd, v, best)
            bidx = jnp.where(upd, i * L + lane, bidx)
            return best, bidx

        best, bidx = lax.fori_loop(
            0, S // L, body,
            (jnp.full((L,), -jnp.inf, jnp.float32), jnp.zeros((L,), jnp.int32)),
        )
        m = jnp.float32(-jnp.inf)
        a = jnp.int32(S)
        for l in range(L):
            vl = best[l]
            il = bidx[l]
            take = (vl > m) | ((vl == m) & (il < a))
            m = jnp.where(take, vl, m)
            a = jnp.where(take, il, a)

        for p in range(N_SAMPLES):
            q = a + p
            q = jnp.where(q >= S, q - S, q)
            blk = (q // L) * L
            buf_v[r, p, pl.ds(blk, L)] = jnp.where(
                lane == q - blk, m, jnp.float32(0.0))

    for i in range(len(pieces)):
        if i > 0:
            reads[i].wait()
            issue_writes(i)
        if i + NBUF < len(pieces):
            for c in writes[i]:
                c.wait()
            issue_read(i + NBUF)
    for i in range(len(pieces) - NBUF, len(pieces)):
        for c in writes[i]:
            c.wait()

    diag = []
    for r, j in enumerate(rows):
        for p in range(N_SAMPLES):
            diag.append(pltpu.async_copy(
                buf_v.at[r, p], out_hbm.at[j, p, pl.ds(j * S, S)], sem_d))
    for c in diag:
        c.wait()


def kernel(x):
    return _sc_kernel(x)
